# baseline (speedup 1.0000x reference)
import functools

import jax
import jax.numpy as jnp
from jax import lax
from jax.experimental import pallas as pl
from jax.experimental.pallas import tpu as pltpu
from jax.experimental.pallas import tpu_sc as plsc

LTM_SLOTS = 100
VECTOR_DIM = 64
DECAY_RATE = 0.995
IMPORTANCE_THRESHOLD = 0.45
SIMILARITY_THRESHOLD = 0.85
OLD_WEIGHT = 0.8
NEW_WEIGHT = 0.2
BOOST_FACTOR = 0.5
NEG_BIG = -3.4e38
PAD_STRENGTH = 1e9
RPS = 7

IV_OFF = 0
IMP_OFF = 64
STR_OFF = 80
CB_LEN = 192


def _rsqrt16(x):
    i = plsc.bitcast(x, jnp.int32)
    i = jnp.int32(0x5F3759DF) - (i >> 1)
    y = plsc.bitcast(i, jnp.float32)
    for _ in range(3):
        y = y * (jnp.float32(1.5) - jnp.float32(0.5) * x * y * y)
    return y


def _rsqrt_scalar(x):
    return jnp.max(_rsqrt16(jnp.broadcast_to(x, (16,))))


def _ffs(mask):
    lane = plsc.all_reduce_ffs(mask)
    if lane.ndim:
        lane = jnp.max(lane)
    return lane


_mesh = plsc.VectorSubcoreMesh(
    core_axis_name="c", subcore_axis_name="s", num_cores=1)


@functools.partial(
    pl.kernel,
    out_type=(
        jax.ShapeDtypeStruct((LTM_SLOTS * VECTOR_DIM,), jnp.float32),
        jax.ShapeDtypeStruct((LTM_SLOTS,), jnp.float32),
    ),
    mesh=_mesh,
    scratch_types=[
        pltpu.VMEM((CB_LEN,), jnp.float32),
        pltpu.VMEM((RPS * VECTOR_DIM,), jnp.float32),
        pltpu.VMEM((16,), jnp.float32),
        pltpu.VMEM((256,), jnp.float32),
        pltpu.VMEM((VECTOR_DIM,), jnp.float32),
        pltpu.VMEM_SHARED((256,), jnp.float32),
    ],
    compiler_params=pltpu.CompilerParams(needs_layout_passes=False),
)
def _sc_kernel(cb_hbm, ltm_hbm, outm_hbm, outs_hbm,
               cb_v, ltm_s, rec_v, rec_all_v, old_v, shared):
    s = lax.axis_index("s")
    lid = lax.iota(jnp.int32, 16)
    base = s * RPS

    pltpu.sync_copy(cb_hbm, cb_v)

    @pl.when(s <= 13)
    def _():
        pltpu.sync_copy(
            ltm_hbm.at[pl.ds(base * VECTOR_DIM, RPS * VECTOR_DIM)], ltm_s)

    @pl.when(s == 14)
    def _():
        pltpu.sync_copy(ltm_hbm.at[pl.ds(98 * VECTOR_DIM, 2 * VECTOR_DIM)],
                        ltm_s.at[pl.ds(0, 2 * VECTOR_DIM)])

    v = [cb_v[pl.ds(IV_OFF + 16 * j, 16)] for j in range(4)]
    nsv = jnp.sum(v[0] * v[0] + v[1] * v[1] + v[2] * v[2] + v[3] * v[3])
    inv1 = jnp.minimum(_rsqrt_scalar(nsv), jnp.float32(1e12))
    v1 = [vj * inv1 for vj in v]
    nsv1 = nsv * inv1 * inv1
    inv2 = jnp.minimum(_rsqrt_scalar(nsv1), jnp.float32(1e12))
    vn = [vj * inv2 for vj in v1]

    dvec = jnp.zeros((16,), jnp.float32)
    nsvec = jnp.zeros((16,), jnp.float32)
    for k in range(RPS):
        r = [ltm_s[pl.ds(64 * k + 16 * j, 16)] for j in range(4)]
        dacc = r[0] * vn[0] + r[1] * vn[1] + r[2] * vn[2] + r[3] * vn[3]
        nacc = r[0] * r[0] + r[1] * r[1] + r[2] * r[2] + r[3] * r[3]
        klane = lid == k
        dvec = jnp.where(klane, jnp.sum(dacc), dvec)
        nsvec = jnp.where(klane, jnp.sum(nacc), nsvec)

    nrows = jnp.where(s == 14, 2, jnp.where(s == 15, 0, RPS))
    lanemask = lid < nrows
    qvec = dvec * jnp.minimum(_rsqrt16(nsvec), jnp.float32(1e8))
    qvec = jnp.where(lanemask, qvec, jnp.float32(NEG_BIG))
    nsvec = jnp.where(lanemask, nsvec, jnp.float32(0.0))

    local_q = jnp.max(qvec)
    local_i = base + _ffs(qvec == local_q)
    local_ns = jnp.max(nsvec)

    rec = (jnp.where(lid == 0, local_q, jnp.float32(0.0))
           + jnp.where(lid == 1, local_i.astype(jnp.float32), jnp.float32(0.0))
           + jnp.where(lid == 2, local_ns, jnp.float32(0.0)))
    rec_v[...] = rec
    pltpu.sync_copy(rec_v, shared.at[pl.ds(s * 16, 16)])

    @pl.when(s <= 13)
    def _():
        pltpu.sync_copy(
            ltm_s, outm_hbm.at[pl.ds(base * VECTOR_DIM, RPS * VECTOR_DIM)])

    @pl.when(s == 14)
    def _():
        pltpu.sync_copy(ltm_s.at[pl.ds(0, 2 * VECTOR_DIM)],
                        outm_hbm.at[pl.ds(98 * VECTOR_DIM, 2 * VECTOR_DIM)])

    plsc.subcore_barrier()

    @pl.when(s == 0)
    def _():
        pltpu.sync_copy(shared, rec_all_v)
        qv = plsc.load_gather(rec_all_v, [lid * 16])
        iv = plsc.load_gather(rec_all_v, [lid * 16 + 1])
        nsv16 = plsc.load_gather(rec_all_v, [lid * 16 + 2])

        best_q = jnp.max(qv)
        lane = _ffs(qv == best_q)
        best_i = jnp.sum(jnp.where(lid == lane, iv, jnp.float32(0.0))
                         ).astype(jnp.int32)
        max_ns = jnp.max(nsv16)
        imp = jnp.max(cb_v[pl.ds(IMP_OFF, 16)])

        best_s = jnp.float32(3.4e38)
        weak_i = jnp.int32(0)
        for k in range(7):
            sk = cb_v[pl.ds(STR_OFF + 16 * k, 16)]
            cmin = jnp.min(sk)
            ln = _ffs(sk == cmin)
            better = cmin < best_s
            weak_i = jnp.where(better, 16 * k + ln, weak_i)
            best_s = jnp.minimum(best_s, cmin)

        all_empty = max_ns < jnp.float32(1e-12)
        reinforce = jnp.logical_and(
            jnp.logical_not(all_empty),
            best_q > jnp.float32(SIMILARITY_THRESHOLD))
        slot = jnp.where(reinforce, best_i, weak_i)
        store_b = imp > jnp.float32(IMPORTANCE_THRESHOLD)

        pltpu.sync_copy(
            ltm_hbm.at[pl.ds(best_i * VECTOR_DIM, VECTOR_DIM)], old_v)
        old = [old_v[pl.ds(16 * j, 16)] for j in range(4)]
        str_msi = jnp.max(
            plsc.load_gather(cb_v, [jnp.broadcast_to(STR_OFF + best_i, (16,))]))
        boosted = jnp.minimum(str_msi + imp * jnp.float32(BOOST_FACTOR),
                              jnp.float32(1.0))
        new_str = jnp.where(reinforce, boosted, imp)

        merged = [jnp.float32(OLD_WEIGHT) * old[j]
                  + jnp.float32(NEW_WEIGHT) * v1[j] for j in range(4)]
        mns = jnp.sum(merged[0] * merged[0] + merged[1] * merged[1]
                      + merged[2] * merged[2] + merged[3] * merged[3])
        invm = jnp.minimum(_rsqrt_scalar(mns), jnp.float32(1e12))
        slot_vec = [jnp.where(reinforce, merged[j] * invm, v1[j])
                    for j in range(4)]

        @pl.when(store_b)
        def _write():
            for j in range(4):
                old_v[pl.ds(16 * j, 16)] = slot_vec[j]
            pltpu.sync_copy(
                old_v, outm_hbm.at[pl.ds(slot * VECTOR_DIM, VECTOR_DIM)])
            plsc.store_scatter(cb_v, [jnp.broadcast_to(STR_OFF + slot, (16,))],
                               jnp.broadcast_to(new_str, (16,)))

        for k in range(7):
            x = cb_v[pl.ds(STR_OFF + 16 * k, 16)] * jnp.float32(DECAY_RATE)
            x = x * (x > jnp.float32(0.01)).astype(jnp.float32)
            cb_v[pl.ds(STR_OFF + 16 * k, 16)] = x

        pltpu.sync_copy(cb_v.at[pl.ds(STR_OFF, LTM_SLOTS)], outs_hbm)


def kernel(input_vector, importance_score, ltm_matrix, ltm_strengths):
    cb = jnp.concatenate([
        input_vector,
        jnp.full((16,), importance_score, dtype=jnp.float32),
        ltm_strengths,
        jnp.full((CB_LEN - STR_OFF - LTM_SLOTS,), PAD_STRENGTH,
                 dtype=jnp.float32),
    ])
    outm, outs = _sc_kernel(cb, ltm_matrix.reshape(-1))
    return outm.reshape(LTM_SLOTS, VECTOR_DIM), outs


# speedup vs baseline: 1.0103x; 1.0103x over previous
import functools

import jax
import jax.numpy as jnp
from jax import lax
from jax.experimental import pallas as pl
from jax.experimental.pallas import tpu as pltpu
from jax.experimental.pallas import tpu_sc as plsc

LTM_SLOTS = 100
VECTOR_DIM = 64
DECAY_RATE = 0.995
IMPORTANCE_THRESHOLD = 0.45
SIMILARITY_THRESHOLD = 0.85
OLD_WEIGHT = 0.8
NEW_WEIGHT = 0.2
BOOST_FACTOR = 0.5
NEG_BIG = -3.4e38
PAD_STRENGTH = 1e9
RPS = 7

IV_OFF = 0
IMP_OFF = 64
STR_OFF = 80
CB_LEN = 192


def _rsqrt16(x):
    i = plsc.bitcast(x, jnp.int32)
    i = jnp.int32(0x5F3759DF) - (i >> 1)
    y = plsc.bitcast(i, jnp.float32)
    for _ in range(3):
        y = y * (jnp.float32(1.5) - jnp.float32(0.5) * x * y * y)
    return y


def _rsqrt_scalar(x):
    return jnp.max(_rsqrt16(jnp.broadcast_to(x, (16,))))


def _ffs(mask):
    lane = plsc.all_reduce_ffs(mask)
    if lane.ndim:
        lane = jnp.max(lane)
    return lane


_mesh = plsc.VectorSubcoreMesh(
    core_axis_name="c", subcore_axis_name="s", num_cores=1)


@functools.partial(
    pl.kernel,
    out_type=(
        jax.ShapeDtypeStruct((LTM_SLOTS * VECTOR_DIM,), jnp.float32),
        jax.ShapeDtypeStruct((LTM_SLOTS,), jnp.float32),
    ),
    mesh=_mesh,
    scratch_types=[
        pltpu.VMEM((CB_LEN,), jnp.float32),
        pltpu.VMEM((RPS * VECTOR_DIM,), jnp.float32),
        pltpu.VMEM((16,), jnp.float32),
        pltpu.VMEM((256,), jnp.float32),
        pltpu.VMEM((VECTOR_DIM,), jnp.float32),
        pltpu.VMEM_SHARED((256,), jnp.float32),
        pltpu.VMEM_SHARED((16 * VECTOR_DIM,), jnp.float32),
        pltpu.SemaphoreType.DMA,
    ],
    compiler_params=pltpu.CompilerParams(needs_layout_passes=False),
)
def _sc_kernel(cb_hbm, ltm_hbm, outm_hbm, outs_hbm,
               cb_v, ltm_s, rec_v, rec_all_v, old_v, shared, shared_rows,
               str_sem):
    s = lax.axis_index("s")
    lid = lax.iota(jnp.int32, 16)
    base = s * RPS

    pltpu.sync_copy(cb_hbm, cb_v)

    @pl.when(s <= 13)
    def _():
        pltpu.sync_copy(
            ltm_hbm.at[pl.ds(base * VECTOR_DIM, RPS * VECTOR_DIM)], ltm_s)

    @pl.when(s == 14)
    def _():
        pltpu.sync_copy(ltm_hbm.at[pl.ds(98 * VECTOR_DIM, 2 * VECTOR_DIM)],
                        ltm_s.at[pl.ds(0, 2 * VECTOR_DIM)])

    v = [cb_v[pl.ds(IV_OFF + 16 * j, 16)] for j in range(4)]
    nsv = jnp.sum(v[0] * v[0] + v[1] * v[1] + v[2] * v[2] + v[3] * v[3])
    inv1 = jnp.minimum(_rsqrt_scalar(nsv), jnp.float32(1e12))
    v1 = [vj * inv1 for vj in v]
    nsv1 = nsv * inv1 * inv1
    inv2 = jnp.minimum(_rsqrt_scalar(nsv1), jnp.float32(1e12))
    vn = [vj * inv2 for vj in v1]

    dvec = jnp.zeros((16,), jnp.float32)
    nsvec = jnp.zeros((16,), jnp.float32)
    for k in range(RPS):
        r = [ltm_s[pl.ds(64 * k + 16 * j, 16)] for j in range(4)]
        dacc = r[0] * vn[0] + r[1] * vn[1] + r[2] * vn[2] + r[3] * vn[3]
        nacc = r[0] * r[0] + r[1] * r[1] + r[2] * r[2] + r[3] * r[3]
        klane = lid == k
        dvec = jnp.where(klane, jnp.sum(dacc), dvec)
        nsvec = jnp.where(klane, jnp.sum(nacc), nsvec)

    nrows = jnp.where(s == 14, 2, jnp.where(s == 15, 0, RPS))
    lanemask = lid < nrows
    qvec = dvec * jnp.minimum(_rsqrt16(nsvec), jnp.float32(1e8))
    qvec = jnp.where(lanemask, qvec, jnp.float32(NEG_BIG))
    nsvec = jnp.where(lanemask, nsvec, jnp.float32(0.0))

    local_q = jnp.max(qvec)
    local_klocal = _ffs(qvec == local_q)
    local_i = base + local_klocal
    local_ns = jnp.max(nsvec)

    pltpu.sync_copy(ltm_s.at[pl.ds(local_klocal * VECTOR_DIM, VECTOR_DIM)],
                    shared_rows.at[pl.ds(s * VECTOR_DIM, VECTOR_DIM)])

    rec = (jnp.where(lid == 0, local_q, jnp.float32(0.0))
           + jnp.where(lid == 1, local_i.astype(jnp.float32), jnp.float32(0.0))
           + jnp.where(lid == 2, local_ns, jnp.float32(0.0)))
    rec_v[...] = rec
    pltpu.sync_copy(rec_v, shared.at[pl.ds(s * 16, 16)])

    @pl.when(s <= 13)
    def _():
        pltpu.sync_copy(
            ltm_s, outm_hbm.at[pl.ds(base * VECTOR_DIM, RPS * VECTOR_DIM)])

    @pl.when(s == 14)
    def _():
        pltpu.sync_copy(ltm_s.at[pl.ds(0, 2 * VECTOR_DIM)],
                        outm_hbm.at[pl.ds(98 * VECTOR_DIM, 2 * VECTOR_DIM)])

    plsc.subcore_barrier()

    @pl.when(s == 0)
    def _():
        pltpu.sync_copy(shared, rec_all_v)
        qv = plsc.load_gather(rec_all_v, [lid * 16])
        iv = plsc.load_gather(rec_all_v, [lid * 16 + 1])
        nsv16 = plsc.load_gather(rec_all_v, [lid * 16 + 2])

        best_q = jnp.max(qv)
        lane = _ffs(qv == best_q)
        best_i = jnp.sum(jnp.where(lid == lane, iv, jnp.float32(0.0))
                         ).astype(jnp.int32)
        max_ns = jnp.max(nsv16)
        imp = jnp.max(cb_v[pl.ds(IMP_OFF, 16)])

        best_s = jnp.float32(3.4e38)
        weak_i = jnp.int32(0)
        for k in range(7):
            sk = cb_v[pl.ds(STR_OFF + 16 * k, 16)]
            cmin = jnp.min(sk)
            ln = _ffs(sk == cmin)
            better = cmin < best_s
            weak_i = jnp.where(better, 16 * k + ln, weak_i)
            best_s = jnp.minimum(best_s, cmin)

        all_empty = max_ns < jnp.float32(1e-12)
        reinforce = jnp.logical_and(
            jnp.logical_not(all_empty),
            best_q > jnp.float32(SIMILARITY_THRESHOLD))
        slot = jnp.where(reinforce, best_i, weak_i)
        store_b = imp > jnp.float32(IMPORTANCE_THRESHOLD)

        pltpu.sync_copy(
            shared_rows.at[pl.ds(lane * VECTOR_DIM, VECTOR_DIM)], old_v)
        old = [old_v[pl.ds(16 * j, 16)] for j in range(4)]
        str_msi = jnp.max(
            plsc.load_gather(cb_v, [jnp.broadcast_to(STR_OFF + best_i, (16,))]))
        boosted = jnp.minimum(str_msi + imp * jnp.float32(BOOST_FACTOR),
                              jnp.float32(1.0))
        new_str = jnp.where(reinforce, boosted, imp)

        merged = [jnp.float32(OLD_WEIGHT) * old[j]
                  + jnp.float32(NEW_WEIGHT) * v1[j] for j in range(4)]
        mns = jnp.sum(merged[0] * merged[0] + merged[1] * merged[1]
                      + merged[2] * merged[2] + merged[3] * merged[3])
        invm = jnp.minimum(_rsqrt_scalar(mns), jnp.float32(1e12))
        slot_vec = [jnp.where(reinforce, merged[j] * invm, v1[j])
                    for j in range(4)]

        @pl.when(store_b)
        def _write():
            for j in range(4):
                old_v[pl.ds(16 * j, 16)] = slot_vec[j]
            plsc.store_scatter(cb_v, [jnp.broadcast_to(STR_OFF + slot, (16,))],
                               jnp.broadcast_to(new_str, (16,)))

        for k in range(7):
            x = cb_v[pl.ds(STR_OFF + 16 * k, 16)] * jnp.float32(DECAY_RATE)
            x = x * (x > jnp.float32(0.01)).astype(jnp.float32)
            cb_v[pl.ds(STR_OFF + 16 * k, 16)] = x

        h_str = pltpu.async_copy(
            cb_v.at[pl.ds(STR_OFF, LTM_SLOTS)], outs_hbm, str_sem)

        @pl.when(store_b)
        def _patch():
            pltpu.sync_copy(
                old_v, outm_hbm.at[pl.ds(slot * VECTOR_DIM, VECTOR_DIM)])

        h_str.wait()


def kernel(input_vector, importance_score, ltm_matrix, ltm_strengths):
    cb = jnp.concatenate([
        input_vector,
        jnp.full((16,), importance_score, dtype=jnp.float32),
        ltm_strengths,
        jnp.full((CB_LEN - STR_OFF - LTM_SLOTS,), PAD_STRENGTH,
                 dtype=jnp.float32),
    ])
    outm, outs = _sc_kernel(cb, ltm_matrix.reshape(-1))
    return outm.reshape(LTM_SLOTS, VECTOR_DIM), outs
